# baseline (speedup 1.0000x reference)
import dataclasses
import functools

import jax
import jax.numpy as jnp
from jax import lax
from jax.experimental import pallas as pl
from jax.experimental.pallas import tpu as pltpu
from jax.experimental.pallas import tpu_sc as plsc

N0, E0 = 10000, 320000
N1, E1 = 2500, 40000
N2, E2 = 625, 10000
D = 128

_NC, _NS, _L = 2, 16, 16
_NW = _NC * _NS
_CH = 128

P0, P1, P2 = 10240, 2560, 640

HB = 16384
O_D0, O_D1, O_C0, O_D2, O_C1 = 0, 10000, 12500, 15000, 15625
O_PAD = 16350

_mesh = plsc.VectorSubcoreMesh(core_axis_name="c", subcore_axis_name="s")

_sc_params = pltpu.CompilerParams()
if "needs_layout_passes" in pltpu.CompilerParams.__dataclass_fields__:
    _sc_params = dataclasses.replace(_sc_params, needs_layout_passes=False)


def _pad1(a, n, val):
    return jnp.concatenate([a, jnp.full((n - a.shape[0],), val, a.dtype)])


def _slices(rows_per, mx=128):
    out, off = [], 0
    while off < rows_per:
        sz = min(mx, rows_per - off)
        out.append((off, sz))
        off += sz
    return out


def _sc_hist(hidx):
    e = hidx.shape[0]
    ew = e // _NW
    nch = ew // 752
    assert ew % 752 == 0

    @functools.partial(
        pl.kernel,
        out_type=jax.ShapeDtypeStruct((_NC, 128, 128), jnp.float32),
        mesh=_mesh,
        compiler_params=_sc_params,
        scratch_types=[
            pltpu.VMEM((752,), jnp.int32),
            pltpu.VMEM((128, 128), jnp.float32),
            pltpu.VMEM((128,), jnp.int32),
            pltpu.VMEM_SHARED((128, 128), jnp.float32),
            pltpu.SemaphoreType.DMA,
        ],
    )
    def k(hidx_hbm, out_hbm, idxb, lhist, rowid, shist, sem):
        c = lax.axis_index("c")
        s = lax.axis_index("s")
        w = c * _NS + s
        zeros = jnp.zeros((_L,), jnp.float32)

        @pl.loop(0, 128)
        def _(r):
            @pl.loop(0, 8)
            def _(q):
                lhist[r, pl.ds(q * _L, _L)] = zeros

        @pl.loop(0, 8)
        def _(q):
            rowid[pl.ds(q * _L, _L)] = (
                lax.iota(jnp.int32, _L) + q * _L
            )

        pltpu.sync_copy(lhist.at[pl.ds(s * 8, 8)], shist.at[pl.ds(s * 8, 8)])
        plsc.subcore_barrier()

        ones = jnp.ones((_L,), jnp.float32)

        @pl.loop(0, nch)
        def _(j):
            base = w * ew + j * 752
            pltpu.sync_copy(hidx_hbm.at[pl.ds(base, 752)], idxb)

            @pl.loop(0, 752 // _L)
            def _(q):
                iv = idxb[pl.ds(q * _L, _L)]
                rv = jax.lax.shift_right_logical(iv, 7)
                cv = jax.lax.bitwise_and(iv, 127)
                plsc.addupdate_scatter(lhist, [rv, cv], ones)

        pltpu.sync_copy(lhist, shist.at[rowid], add=True)
        plsc.subcore_barrier()
        pltpu.sync_copy(shist.at[pl.ds(s * 8, 8)],
                        out_hbm.at[c].at[pl.ds(s * 8, 8)])

    return k(hidx).reshape(_NC, HB)


def _sc_edge_scatter(table, src_idx, dst_idx, p_out):
    e = src_idx.shape[0]
    ew = e // _NW
    nch = ew // _CH
    rows_per = p_out // _NS

    @functools.partial(
        pl.kernel,
        out_type=jax.ShapeDtypeStruct((_NC, p_out, D), jnp.float32),
        mesh=_mesh,
        compiler_params=_sc_params,
        scratch_types=[
            pltpu.VMEM((_CH,), jnp.int32),
            pltpu.VMEM((_CH,), jnp.int32),
            pltpu.VMEM((_CH, D), jnp.float32),
            pltpu.VMEM((128, D), jnp.float32),
            pltpu.VMEM_SHARED((p_out, D), jnp.float32),
            pltpu.SemaphoreType.DMA,
        ],
    )
    def k(table_hbm, src_hbm, dst_hbm, out_hbm, sidx, didx, rows, zbuf, acc,
          sem):
        c = lax.axis_index("c")
        s = lax.axis_index("s")
        w = c * _NS + s
        zeros = jnp.zeros((_L,), jnp.float32)

        @pl.loop(0, 128)
        def _(r):
            @pl.loop(0, D // _L)
            def _(q):
                zbuf[r, pl.ds(q * _L, _L)] = zeros

        base_rows = s * rows_per
        for off, sz in _slices(rows_per):
            pltpu.sync_copy(zbuf.at[pl.ds(0, sz)],
                            acc.at[pl.ds(base_rows + off, sz)])
        plsc.subcore_barrier()

        @pl.loop(0, nch)
        def _(j):
            base = w * ew + j * _CH
            pltpu.sync_copy(src_hbm.at[pl.ds(base, _CH)], sidx)
            pltpu.sync_copy(dst_hbm.at[pl.ds(base, _CH)], didx)
            pltpu.async_copy(table_hbm.at[sidx], rows, sem).wait()
            pltpu.sync_copy(rows, acc.at[didx], add=True)

        plsc.subcore_barrier()
        for off, sz in _slices(rows_per):
            pltpu.sync_copy(acc.at[pl.ds(base_rows + off, sz)],
                            out_hbm.at[c].at[pl.ds(base_rows + off, sz)])

    return k(table, src_idx, dst_idx)


def _sc_edge_scatter_g(table, src2d, dst2d, p_out):
    e = src2d.shape[0] * 128
    ew = e // _NW
    nch = ew // _CH
    assert nch % 8 == 0
    rows_per = p_out // _NS

    @functools.partial(
        pl.kernel,
        out_type=jax.ShapeDtypeStruct((_NC, p_out, D), jnp.float32),
        mesh=_mesh,
        compiler_params=_sc_params,
        scratch_types=[
            pltpu.VMEM((8, _CH), jnp.int32),
            pltpu.VMEM((8, _CH), jnp.int32),
            pltpu.VMEM((_CH, D), jnp.float32),
            pltpu.VMEM((64, D), jnp.float32),
            pltpu.VMEM_SHARED((p_out, D), jnp.float32),
            pltpu.SemaphoreType.DMA,
        ],
    )
    def k(table_hbm, src_hbm, dst_hbm, out_hbm, sidx, didx, rows, zbuf, acc,
          sem):
        c = lax.axis_index("c")
        s = lax.axis_index("s")
        w = c * _NS + s
        zeros = jnp.zeros((_L,), jnp.float32)

        @pl.loop(0, 64)
        def _(r):
            @pl.loop(0, D // _L)
            def _(q):
                zbuf[r, pl.ds(q * _L, _L)] = zeros

        base_rows = s * rows_per
        for off, sz in _slices(rows_per, 64):
            pltpu.sync_copy(zbuf.at[pl.ds(0, sz)],
                            acc.at[pl.ds(base_rows + off, sz)])
        plsc.subcore_barrier()

        rw = nch
        @pl.loop(0, nch // 8)
        def _(g):
            rbase = w * rw + g * 8
            pltpu.sync_copy(src_hbm.at[pl.ds(rbase, 8), :], sidx)
            pltpu.sync_copy(dst_hbm.at[pl.ds(rbase, 8), :], didx)
            for j in range(8):
                pltpu.async_copy(table_hbm.at[sidx.at[j]], rows, sem).wait()
                pltpu.sync_copy(rows, acc.at[didx.at[j]], add=True)

        plsc.subcore_barrier()
        for off, sz in _slices(rows_per):
            pltpu.sync_copy(acc.at[pl.ds(base_rows + off, sz)],
                            out_hbm.at[c].at[pl.ds(base_rows + off, sz)])

    return k(table, src2d, dst2d)


def _sc_gather(table, idx):
    e = idx.shape[0]
    ew = e // _NW
    nch = ew // _CH

    @functools.partial(
        pl.kernel,
        out_type=jax.ShapeDtypeStruct((e, D), jnp.float32),
        mesh=_mesh,
        compiler_params=_sc_params,
        scratch_types=[
            pltpu.VMEM((_CH,), jnp.int32),
            pltpu.VMEM((_CH, D), jnp.float32),
            pltpu.SemaphoreType.DMA,
        ],
    )
    def k(table_hbm, idx_hbm, out_hbm, idxb, rows, sem):
        c = lax.axis_index("c")
        s = lax.axis_index("s")
        w = c * _NS + s

        @pl.loop(0, nch)
        def _(j):
            base = w * ew + j * _CH
            pltpu.sync_copy(idx_hbm.at[pl.ds(base, _CH)], idxb)
            pltpu.async_copy(table_hbm.at[idxb], rows, sem).wait()
            pltpu.sync_copy(rows, out_hbm.at[pl.ds(base, _CH)])

    return k(table, idx)


def _dot(a, w):
    return lax.dot_general(a, w, (((1,), (0,)), ((), ())),
                           precision=lax.Precision.HIGHEST,
                           preferred_element_type=jnp.float32)


def _tc_conv(acc, s, w, b, relu):
    p = acc.shape[1]

    def body(a_ref, s_ref, w_ref, b_ref, o_ref):
        a = (a_ref[0] + a_ref[1]) * s_ref[...]
        y = _dot(a, w_ref[...]) + b_ref[...]
        o_ref[...] = jnp.maximum(y, 0.0) if relu else y

    return pl.pallas_call(
        body, out_shape=jax.ShapeDtypeStruct((p, D), jnp.float32),
    )(acc, s, w.reshape(D, D), b.reshape(1, D))


def _tc_scale_rows(acc, s):
    p = acc.shape[1]

    def body(a_ref, s_ref, o_ref):
        o_ref[...] = (a_ref[0] + a_ref[1]) * s_ref[...]

    return pl.pallas_call(
        body, out_shape=jax.ShapeDtypeStruct((p, D), jnp.float32),
    )(acc, s)


def _tc_merge(g, xskip, s):
    p = xskip.shape[0]

    def body(g_ref, x_ref, s_ref, o_ref):
        o_ref[...] = (g_ref[...] + x_ref[...]) * s_ref[...]

    return pl.pallas_call(
        body,
        grid=(1,),
        in_specs=[
            pl.BlockSpec((p, D), lambda i: (0, 0)),
            pl.BlockSpec((p, D), lambda i: (0, 0)),
            pl.BlockSpec((p, 1), lambda i: (0, 0)),
        ],
        out_specs=pl.BlockSpec((p, D), lambda i: (0, 0)),
        out_shape=jax.ShapeDtypeStruct((p, D), jnp.float32),
    )(g, xskip, s)


def _tc_scales(d0, d1, c0, d2, c1, x):
    def body(d0_ref, d1_ref, c0_ref, d2_ref, c1_ref, x_ref,
             s0_ref, s1i_ref, s1o_ref, s2i_ref, s2o_ref, xs0_ref):
        rs0 = lax.rsqrt(jnp.maximum(d0_ref[0] + d0_ref[1], 1.0))
        s0_ref[...] = jnp.concatenate(
            [rs0, jnp.ones((P0 - N0, 1), jnp.float32)], axis=0)
        rs1 = lax.rsqrt(jnp.maximum(d1_ref[0] + d1_ref[1], 1.0))
        s1o_ref[...] = rs1
        s1i_ref[...] = rs1 / jnp.maximum(c0_ref[0] + c0_ref[1], 1.0)
        rs2 = lax.rsqrt(jnp.maximum(d2_ref[0] + d2_ref[1], 1.0))
        s2o_ref[...] = rs2
        s2i_ref[...] = rs2 / jnp.maximum(c1_ref[0] + c1_ref[1], 1.0)
        xs0_ref[...] = x_ref[...] * rs0

    return pl.pallas_call(
        body,
        out_shape=(
            jax.ShapeDtypeStruct((P0, 1), jnp.float32),
            jax.ShapeDtypeStruct((P1, 1), jnp.float32),
            jax.ShapeDtypeStruct((P1, 1), jnp.float32),
            jax.ShapeDtypeStruct((P2, 1), jnp.float32),
            jax.ShapeDtypeStruct((P2, 1), jnp.float32),
            jax.ShapeDtypeStruct((N0, D), jnp.float32),
        ),
    )(d0, d1, c0, d2, c1, x)


def kernel(x, edge_index_0, edge_index_1, edge_index_2, clusters_0, clusters_1,
           batch, W_d0, b_d0, W_d1, b_d1, W_bot, b_bot, W_u1, b_u1, W_u0, b_u0):
    i32 = jnp.int32

    def _padd(a, n, base, span):
        pad = n - a.shape[0]
        fill = base + lax.rem(lax.iota(i32, pad), jnp.full((pad,), span, i32))
        return jnp.concatenate([a, fill])

    e0p, e1p, e2p = 327680, 40960, 12288
    src0 = _pad1(edge_index_0[0], e0p, 0)
    dst0 = _padd(edge_index_0[1], e0p, N0, P0 - N0)
    src1 = _pad1(edge_index_1[0], e1p, 0)
    dst1 = _padd(edge_index_1[1], e1p, N1, P1 - N1)
    src2 = _pad1(edge_index_2[0], e2p, 0)
    dst2 = _padd(edge_index_2[1], e2p, N2, P2 - N2)

    pool0_src = _pad1(lax.iota(i32, N0), 12288, 0)
    pool0_dst = _padd(clusters_0, 12288, N1, P1 - N1)
    pool1_src = _pad1(lax.iota(i32, N1), 4096, 0)
    pool1_dst = _padd(clusters_1, 4096, N2, P2 - N2)
    up1_idx = _pad1(clusters_1, 4096, 0)
    up0_idx = _pad1(clusters_0, 12288, 0)

    hidx = jnp.concatenate([
        edge_index_0[1],
        edge_index_1[1] + O_D1,
        clusters_0 + O_C0,
        edge_index_2[1] + O_D2,
        clusters_1 + O_C1,
    ])
    hidx = _pad1(hidx, 385024, O_PAD)
    src0_2d = src0.reshape(-1, 128)
    dst0_2d = dst0.reshape(-1, 128)

    hist = _sc_hist(hidx)
    d0 = hist[:, O_D0:O_D0 + N0, None]
    d1 = hist[:, O_D1:O_D1 + P1, None]
    c0 = hist[:, O_C0:O_C0 + P1, None]
    d2 = hist[:, O_D2:O_D2 + P2, None]
    c1 = hist[:, O_C1:O_C1 + P2, None]

    s0, s1i, s1o, s2i, s2o, xs0 = _tc_scales(d0, d1, c0, d2, c1, x)

    acc = _sc_edge_scatter_g(xs0, src0_2d, dst0_2d, P0)
    x0 = _tc_conv(acc, s0, W_d0, b_d0, relu=True)

    accp1 = _sc_edge_scatter(x0, pool0_src, pool0_dst, P1)
    xs1 = _tc_scale_rows(accp1, s1i)
    acc1 = _sc_edge_scatter(xs1, src1, dst1, P1)
    x1 = _tc_conv(acc1, s1o, W_d1, b_d1, relu=True)

    accp2 = _sc_edge_scatter(x1, pool1_src, pool1_dst, P2)
    xs2 = _tc_scale_rows(accp2, s2i)
    acc2 = _sc_edge_scatter(xs2, src2, dst2, P2)
    x2 = _tc_conv(acc2, s2o, W_bot, b_bot, relu=True)

    g1 = _sc_gather(x2, up1_idx)
    xsu1 = _tc_merge(g1, x1, s1o)
    acc3 = _sc_edge_scatter(xsu1, src1, dst1, P1)
    x1u = _tc_conv(acc3, s1o, W_u1, b_u1, relu=True)

    g0 = _sc_gather(x1u, up0_idx)
    xsu0 = _tc_merge(g0, x0, s0)
    acc4 = _sc_edge_scatter_g(xsu0, src0_2d, dst0_2d, P0)
    out = _tc_conv(acc4, s0, W_u0, b_u0, relu=False)

    return out[:N0]


# speedup vs baseline: 1.2737x; 1.2737x over previous
import dataclasses
import functools

import jax
import jax.numpy as jnp
from jax import lax
from jax.experimental import pallas as pl
from jax.experimental.pallas import tpu as pltpu
from jax.experimental.pallas import tpu_sc as plsc

N0, E0 = 10000, 320000
N1, E1 = 2500, 40000
N2, E2 = 625, 10000
D = 128

_NC, _NS, _L = 2, 16, 16
_NW = _NC * _NS
_CH = 128

P0, P1, P2 = 10240, 2560, 640

HB = 16384
O_D0, O_D1, O_C0, O_D2, O_C1 = 0, 10000, 12500, 15000, 15625
O_PAD = 16350

_mesh = plsc.VectorSubcoreMesh(core_axis_name="c", subcore_axis_name="s")

_sc_params = pltpu.CompilerParams()
if "needs_layout_passes" in pltpu.CompilerParams.__dataclass_fields__:
    _sc_params = dataclasses.replace(_sc_params, needs_layout_passes=False)


def _pad1(a, n, val):
    return jnp.concatenate([a, jnp.full((n - a.shape[0],), val, a.dtype)])


def _slices(rows_per):
    out, off = [], 0
    while off < rows_per:
        sz = min(128, rows_per - off)
        out.append((off, sz))
        off += sz
    return out


def _sc_hist(hidx):
    e = hidx.shape[0]
    ew = e // _NW
    nch = ew // 752
    assert ew % 752 == 0

    @functools.partial(
        pl.kernel,
        out_type=jax.ShapeDtypeStruct((_NC, 128, 128), jnp.float32),
        mesh=_mesh,
        compiler_params=_sc_params,
        scratch_types=[
            pltpu.VMEM((752,), jnp.int32),
            pltpu.VMEM((128, 128), jnp.float32),
            pltpu.VMEM((128,), jnp.int32),
            pltpu.VMEM_SHARED((128, 128), jnp.float32),
            pltpu.SemaphoreType.DMA,
        ],
    )
    def k(hidx_hbm, out_hbm, idxb, lhist, rowid, shist, sem):
        c = lax.axis_index("c")
        s = lax.axis_index("s")
        w = c * _NS + s
        zeros = jnp.zeros((_L,), jnp.float32)

        @pl.loop(0, 128)
        def _(r):
            @pl.loop(0, 8)
            def _(q):
                lhist[r, pl.ds(q * _L, _L)] = zeros

        @pl.loop(0, 8)
        def _(q):
            rowid[pl.ds(q * _L, _L)] = (
                lax.iota(jnp.int32, _L) + q * _L
            )

        pltpu.sync_copy(lhist.at[pl.ds(s * 8, 8)], shist.at[pl.ds(s * 8, 8)])
        plsc.subcore_barrier()

        ones = jnp.ones((_L,), jnp.float32)

        @pl.loop(0, nch)
        def _(j):
            base = w * ew + j * 752
            pltpu.sync_copy(hidx_hbm.at[pl.ds(base, 752)], idxb)

            @pl.loop(0, 752 // _L)
            def _(q):
                iv = idxb[pl.ds(q * _L, _L)]
                rv = jax.lax.shift_right_logical(iv, 7)
                cv = jax.lax.bitwise_and(iv, 127)
                plsc.addupdate_scatter(lhist, [rv, cv], ones)

        pltpu.sync_copy(lhist, shist.at[rowid], add=True)
        plsc.subcore_barrier()
        pltpu.sync_copy(shist.at[pl.ds(s * 8, 8)],
                        out_hbm.at[c].at[pl.ds(s * 8, 8)])

    return k(hidx).reshape(_NC, HB)


def _sc_edge_scatter(table, src_idx, dst_idx, p_out):
    e = src_idx.shape[0]
    ew = e // _NW
    nch = ew // _CH
    rows_per = p_out // _NS

    @functools.partial(
        pl.kernel,
        out_type=jax.ShapeDtypeStruct((_NC, p_out, D), jnp.float32),
        mesh=_mesh,
        compiler_params=_sc_params,
        scratch_types=[
            pltpu.VMEM((_CH,), jnp.int32),
            pltpu.VMEM((_CH,), jnp.int32),
            pltpu.VMEM((_CH, D), jnp.float32),
            pltpu.VMEM((128, D), jnp.float32),
            pltpu.VMEM_SHARED((p_out, D), jnp.float32),
            pltpu.SemaphoreType.DMA,
        ],
    )
    def k(table_hbm, src_hbm, dst_hbm, out_hbm, sidx, didx, rows, zbuf, acc,
          sem):
        c = lax.axis_index("c")
        s = lax.axis_index("s")
        w = c * _NS + s
        zeros = jnp.zeros((_L,), jnp.float32)

        @pl.loop(0, 128)
        def _(r):
            @pl.loop(0, D // _L)
            def _(q):
                zbuf[r, pl.ds(q * _L, _L)] = zeros

        base_rows = s * rows_per
        for off, sz in _slices(rows_per):
            pltpu.sync_copy(zbuf.at[pl.ds(0, sz)],
                            acc.at[pl.ds(base_rows + off, sz)])
        plsc.subcore_barrier()

        @pl.loop(0, nch)
        def _(j):
            base = w * ew + j * _CH
            pltpu.sync_copy(src_hbm.at[pl.ds(base, _CH)], sidx)
            pltpu.sync_copy(dst_hbm.at[pl.ds(base, _CH)], didx)
            pltpu.async_copy(table_hbm.at[sidx], rows, sem).wait()
            pltpu.sync_copy(rows, acc.at[didx], add=True)

        plsc.subcore_barrier()
        for off, sz in _slices(rows_per):
            pltpu.sync_copy(acc.at[pl.ds(base_rows + off, sz)],
                            out_hbm.at[c].at[pl.ds(base_rows + off, sz)])

    return k(table, src_idx, dst_idx)


def _sc_gather(table, idx):
    e = idx.shape[0]
    ew = e // _NW
    nch = ew // _CH

    @functools.partial(
        pl.kernel,
        out_type=jax.ShapeDtypeStruct((e, D), jnp.float32),
        mesh=_mesh,
        compiler_params=_sc_params,
        scratch_types=[
            pltpu.VMEM((_CH,), jnp.int32),
            pltpu.VMEM((_CH, D), jnp.float32),
            pltpu.SemaphoreType.DMA,
        ],
    )
    def k(table_hbm, idx_hbm, out_hbm, idxb, rows, sem):
        c = lax.axis_index("c")
        s = lax.axis_index("s")
        w = c * _NS + s

        @pl.loop(0, nch)
        def _(j):
            base = w * ew + j * _CH
            pltpu.sync_copy(idx_hbm.at[pl.ds(base, _CH)], idxb)
            pltpu.async_copy(table_hbm.at[idxb], rows, sem).wait()
            pltpu.sync_copy(rows, out_hbm.at[pl.ds(base, _CH)])

    return k(table, idx)


def _dot(a, w):
    return lax.dot_general(a, w, (((1,), (0,)), ((), ())),
                           precision=lax.Precision.HIGHEST,
                           preferred_element_type=jnp.float32)


def _tc_conv(acc, s, w, b, relu):
    p = acc.shape[1]

    def body(a_ref, s_ref, w_ref, b_ref, o_ref):
        a = (a_ref[0] + a_ref[1]) * s_ref[...]
        y = _dot(a, w_ref[...]) + b_ref[...]
        o_ref[...] = jnp.maximum(y, 0.0) if relu else y

    return pl.pallas_call(
        body, out_shape=jax.ShapeDtypeStruct((p, D), jnp.float32),
    )(acc, s, w.reshape(D, D), b.reshape(1, D))


def _tc_scale_rows(acc, s):
    p = acc.shape[1]

    def body(a_ref, s_ref, o_ref):
        o_ref[...] = (a_ref[0] + a_ref[1]) * s_ref[...]

    return pl.pallas_call(
        body, out_shape=jax.ShapeDtypeStruct((p, D), jnp.float32),
    )(acc, s)


def _tc_merge(g, xskip, s):
    p = xskip.shape[0]

    def body(g_ref, x_ref, s_ref, o_ref):
        o_ref[...] = (g_ref[...] + x_ref[...]) * s_ref[...]

    return pl.pallas_call(
        body,
        grid=(1,),
        in_specs=[
            pl.BlockSpec((p, D), lambda i: (0, 0)),
            pl.BlockSpec((p, D), lambda i: (0, 0)),
            pl.BlockSpec((p, 1), lambda i: (0, 0)),
        ],
        out_specs=pl.BlockSpec((p, D), lambda i: (0, 0)),
        out_shape=jax.ShapeDtypeStruct((p, D), jnp.float32),
    )(g, xskip, s)


def _tc_scales(d0, d1, c0, d2, c1, x):
    def body(d0_ref, d1_ref, c0_ref, d2_ref, c1_ref, x_ref,
             s0_ref, s1i_ref, s1o_ref, s2i_ref, s2o_ref, xs0_ref):
        rs0 = lax.rsqrt(jnp.maximum(d0_ref[0] + d0_ref[1], 1.0))
        s0_ref[...] = jnp.concatenate(
            [rs0, jnp.ones((P0 - N0, 1), jnp.float32)], axis=0)
        rs1 = lax.rsqrt(jnp.maximum(d1_ref[0] + d1_ref[1], 1.0))
        s1o_ref[...] = rs1
        s1i_ref[...] = rs1 / jnp.maximum(c0_ref[0] + c0_ref[1], 1.0)
        rs2 = lax.rsqrt(jnp.maximum(d2_ref[0] + d2_ref[1], 1.0))
        s2o_ref[...] = rs2
        s2i_ref[...] = rs2 / jnp.maximum(c1_ref[0] + c1_ref[1], 1.0)
        xs0_ref[...] = x_ref[...] * rs0

    return pl.pallas_call(
        body,
        out_shape=(
            jax.ShapeDtypeStruct((P0, 1), jnp.float32),
            jax.ShapeDtypeStruct((P1, 1), jnp.float32),
            jax.ShapeDtypeStruct((P1, 1), jnp.float32),
            jax.ShapeDtypeStruct((P2, 1), jnp.float32),
            jax.ShapeDtypeStruct((P2, 1), jnp.float32),
            jax.ShapeDtypeStruct((N0, D), jnp.float32),
        ),
    )(d0, d1, c0, d2, c1, x)


def kernel(x, edge_index_0, edge_index_1, edge_index_2, clusters_0, clusters_1,
           batch, W_d0, b_d0, W_d1, b_d1, W_bot, b_bot, W_u1, b_u1, W_u0, b_u0):
    i32 = jnp.int32

    def _padd(a, n, base, span):
        pad = n - a.shape[0]
        fill = base + lax.rem(lax.iota(i32, pad), jnp.full((pad,), span, i32))
        return jnp.concatenate([a, fill])

    e0p, e1p, e2p = 323584, 40960, 12288
    src0 = _pad1(edge_index_0[0], e0p, 0)
    dst0 = _padd(edge_index_0[1], e0p, N0, P0 - N0)
    src1 = _pad1(edge_index_1[0], e1p, 0)
    dst1 = _padd(edge_index_1[1], e1p, N1, P1 - N1)
    src2 = _pad1(edge_index_2[0], e2p, 0)
    dst2 = _padd(edge_index_2[1], e2p, N2, P2 - N2)

    pool0_src = _pad1(lax.iota(i32, N0), 12288, 0)
    pool0_dst = _padd(clusters_0, 12288, N1, P1 - N1)
    pool1_src = _pad1(lax.iota(i32, N1), 4096, 0)
    pool1_dst = _padd(clusters_1, 4096, N2, P2 - N2)
    up1_idx = _pad1(clusters_1, 4096, 0)
    up0_idx = _pad1(clusters_0, 12288, 0)

    hidx = jnp.concatenate([
        edge_index_0[1],
        edge_index_1[1] + O_D1,
        clusters_0 + O_C0,
        edge_index_2[1] + O_D2,
        clusters_1 + O_C1,
    ])
    hidx = _pad1(hidx, 385024, O_PAD)

    hist = _sc_hist(hidx)
    d0 = hist[:, O_D0:O_D0 + N0, None]
    d1 = hist[:, O_D1:O_D1 + P1, None]
    c0 = hist[:, O_C0:O_C0 + P1, None]
    d2 = hist[:, O_D2:O_D2 + P2, None]
    c1 = hist[:, O_C1:O_C1 + P2, None]

    s0, s1i, s1o, s2i, s2o, xs0 = _tc_scales(d0, d1, c0, d2, c1, x)

    acc = _sc_edge_scatter(xs0, src0, dst0, P0)
    x0 = _tc_conv(acc, s0, W_d0, b_d0, relu=True)

    accp1 = _sc_edge_scatter(x0, pool0_src, pool0_dst, P1)
    xs1 = _tc_scale_rows(accp1, s1i)
    acc1 = _sc_edge_scatter(xs1, src1, dst1, P1)
    x1 = _tc_conv(acc1, s1o, W_d1, b_d1, relu=True)

    accp2 = _sc_edge_scatter(x1, pool1_src, pool1_dst, P2)
    xs2 = _tc_scale_rows(accp2, s2i)
    acc2 = _sc_edge_scatter(xs2, src2, dst2, P2)
    x2 = _tc_conv(acc2, s2o, W_bot, b_bot, relu=True)

    g1 = _sc_gather(x2, up1_idx)
    xsu1 = _tc_merge(g1, x1, s1o)
    acc3 = _sc_edge_scatter(xsu1, src1, dst1, P1)
    x1u = _tc_conv(acc3, s1o, W_u1, b_u1, relu=True)

    g0 = _sc_gather(x1u, up0_idx)
    xsu0 = _tc_merge(g0, x0, s0)
    acc4 = _sc_edge_scatter(xsu0, src0, dst0, P0)
    out = _tc_conv(acc4, s0, W_u0, b_u0, relu=False)

    return out[:N0]
